# baseline (speedup 1.0000x reference)
import functools

import jax
import jax.numpy as jnp
from jax import lax
from jax.experimental import pallas as pl
from jax.experimental.pallas import tpu as pltpu
from jax.experimental.pallas import tpu_sc as plsc

_LAMBDA = 1.0
_NEG_INF = float("-inf")
_B = 4096
_D = 1024
_E = 8
_BLK = 256
_NBLK = 40
_PAD = _NBLK * _BLK


def _gate_route_kernel(x_ref, wg_ref, bg_ref, meta_ref, be_ref, cv_ref):
    idx8 = jax.lax.broadcasted_iota(jnp.int32, (_B, _E), 1)
    logits = jax.lax.dot_general(
        x_ref[...], wg_ref[...], (((1,), (1,)), ((), ())),
        preferred_element_type=jnp.float32) + bg_ref[...]
    m1 = jnp.max(logits, axis=1, keepdims=True)
    i1 = jnp.min(jnp.where(logits == m1, idx8, _E), axis=1, keepdims=True)
    sel1 = idx8 == i1
    masked = jnp.where(sel1, _NEG_INF, logits)
    m2 = jnp.max(masked, axis=1, keepdims=True)
    i2 = jnp.min(jnp.where(masked == m2, idx8, _E), axis=1, keepdims=True)
    sel2 = idx8 == i2
    e2v = jnp.exp(m2 - m1)
    z = 1.0 + e2v
    p1 = 1.0 / z
    p2 = e2v / z

    pfull = jnp.where(sel1, p1, jnp.where(sel2, p2, 0.0))
    load = jnp.sum(pfull, axis=0, keepdims=True)
    mean = jnp.sum(load) / float(_E)
    var = jnp.sum((load - mean) ** 2) / float(_E - 1)
    cv_ref[...] = jnp.full((8, 128), jnp.sqrt(var) / mean, jnp.float32)

    sel = (sel1 | sel2).astype(jnp.float32)
    n = 512
    r = jax.lax.broadcasted_iota(jnp.int32, (n, n), 0)
    c = jax.lax.broadcasted_iota(jnp.int32, (n, n), 1)
    ltri = (r > c).astype(jnp.float32)
    carry = jnp.zeros((1, _E), jnp.float32)
    chunks = []
    for b in range(_B // n):
        blk = sel[b * n:(b + 1) * n, :]
        chunks.append(carry + jax.lax.dot_general(
            ltri, blk, (((1,), (0,)), ((), ())),
            preferred_element_type=jnp.float32))
        carry = carry + jnp.sum(blk, axis=0, keepdims=True)
    cntb = jnp.concatenate(chunks, axis=0)

    cnt = carry
    pc = jnp.floor((cnt + float(_BLK - 1)) / float(_BLK)) * float(_BLK)
    u = jax.lax.broadcasted_iota(jnp.int32, (_E, _E), 0)
    v = jax.lax.broadcasted_iota(jnp.int32, (_E, _E), 1)
    ustrict = (u < v).astype(jnp.float32)
    off = jax.lax.dot_general(pc, ustrict, (((1,), (0,)), ((), ())),
                              preferred_element_type=jnp.float32)

    sfull = off + cntb
    slot1 = jnp.sum(jnp.where(sel1, sfull, 0.0), axis=1, keepdims=True)
    slot2 = jnp.sum(jnp.where(sel2, sfull, 0.0), axis=1, keepdims=True)
    meta_ref[...] = jnp.concatenate(
        [p1, p2, slot1, slot2, jnp.zeros((_B, 4), jnp.float32)], axis=1)

    endf = off + pc
    jrow = jax.lax.broadcasted_iota(jnp.int32, (_NBLK, _E), 0).astype(
        jnp.float32) * float(_BLK)
    cmp = (jrow >= endf).astype(jnp.float32)
    be = jnp.minimum(jnp.sum(cmp, axis=1, keepdims=True), float(_E - 1))
    be_ref[...] = be


def _gate_route(x, W_gate, b_gate):
    return pl.pallas_call(
        _gate_route_kernel,
        in_specs=[
            pl.BlockSpec((_B, _D), lambda: (0, 0)),
            pl.BlockSpec((_E, _D), lambda: (0, 0)),
            pl.BlockSpec((1, _E), lambda: (0, 0)),
        ],
        out_specs=[
            pl.BlockSpec((_B, _E), lambda: (0, 0)),
            pl.BlockSpec((_NBLK, 1), lambda: (0, 0)),
            pl.BlockSpec((8, 128), lambda: (0, 0)),
        ],
        out_shape=[
            jax.ShapeDtypeStruct((_B, _E), jnp.float32),
            jax.ShapeDtypeStruct((_NBLK, 1), jnp.float32),
            jax.ShapeDtypeStruct((8, 128), jnp.float32),
        ],
    )(x, W_gate, b_gate.reshape(1, _E))


_NW = 32
_TPW = _B // _NW
_GCH = 64


def _sc_gather_kernel(s1_hbm, s2_hbm, x_hbm, xs_hbm,
                      idx_v, rows_v, sem):
    wid = lax.axis_index("s") * 2 + lax.axis_index("c")

    def _chunk(i, _):
        base = wid * _TPW + i * _GCH
        pltpu.sync_copy(x_hbm.at[pl.ds(base, _GCH)], rows_v)
        pltpu.sync_copy(s1_hbm.at[pl.ds(base, _GCH)], idx_v)
        pltpu.async_copy(rows_v, xs_hbm.at[idx_v], sem).wait()
        pltpu.sync_copy(s2_hbm.at[pl.ds(base, _GCH)], idx_v)
        pltpu.async_copy(rows_v, xs_hbm.at[idx_v], sem).wait()
        return 0

    lax.fori_loop(0, _TPW // _GCH, _chunk, 0)


def _sc_gather(slots1, slots2, x):
    mesh = plsc.VectorSubcoreMesh(core_axis_name="c", subcore_axis_name="s")
    fn = pl.kernel(
        _sc_gather_kernel,
        mesh=mesh,
        out_type=jax.ShapeDtypeStruct((_PAD, _D), jnp.float32),
        scratch_types=[
            pltpu.VMEM((_GCH,), jnp.int32),
            pltpu.VMEM((_GCH, _D), jnp.float32),
            pltpu.SemaphoreType.DMA,
        ],
    )
    return fn(slots1, slots2, x)


def _group_mm_kernel(be_ref, xs_ref, w_ref, b_ref, ys_ref):
    xb = xs_ref[...].astype(jnp.bfloat16)
    wb = w_ref[0].astype(jnp.bfloat16)
    ys_ref[...] = jax.lax.dot_general(
        xb, wb, (((1,), (1,)), ((), ())),
        preferred_element_type=jnp.float32) + b_ref[0]


def _group_mm(be, xs, W_experts, b_experts):
    grid_spec = pltpu.PrefetchScalarGridSpec(
        num_scalar_prefetch=1,
        grid=(_NBLK,),
        in_specs=[
            pl.BlockSpec((_BLK, _D), lambda j, be: (j, 0)),
            pl.BlockSpec((1, _D, _D), lambda j, be: (be[j], 0, 0)),
            pl.BlockSpec((1, 1, _D), lambda j, be: (be[j], 0, 0)),
        ],
        out_specs=pl.BlockSpec((_BLK, _D), lambda j, be: (j, 0)),
    )
    return pl.pallas_call(
        _group_mm_kernel,
        grid_spec=grid_spec,
        out_shape=jax.ShapeDtypeStruct((_PAD, _D), jnp.float32),
    )(be, xs, W_experts, b_experts.reshape(_E, 1, _D))


def _sc_combine_gather_kernel(s1_hbm, s2_hbm, ys_hbm, z1_hbm, z2_hbm,
                              idx_v, rows_v, sem):
    wid = lax.axis_index("s") * 2 + lax.axis_index("c")

    def _chunk(args, _):
        s_hbm, z_hbm, i = args
        base = wid * _TPW + i * _GCH
        pltpu.sync_copy(s_hbm.at[pl.ds(base, _GCH)], idx_v)
        pltpu.async_copy(ys_hbm.at[idx_v], rows_v, sem).wait()
        pltpu.sync_copy(rows_v, z_hbm.at[pl.ds(base, _GCH)])

    def _loop1(i, _):
        _chunk((s1_hbm, z1_hbm, i), None)
        return 0

    def _loop2(i, _):
        _chunk((s2_hbm, z2_hbm, i), None)
        return 0

    lax.fori_loop(0, _TPW // _GCH, _loop1, 0)
    lax.fori_loop(0, _TPW // _GCH, _loop2, 0)


def _sc_combine_gather(slots1, slots2, ys):
    mesh = plsc.VectorSubcoreMesh(core_axis_name="c", subcore_axis_name="s")
    fn = pl.kernel(
        _sc_combine_gather_kernel,
        mesh=mesh,
        out_type=[
            jax.ShapeDtypeStruct((_B, _D), jnp.float32),
            jax.ShapeDtypeStruct((_B, _D), jnp.float32),
        ],
        scratch_types=[
            pltpu.VMEM((_GCH,), jnp.int32),
            pltpu.VMEM((_GCH, _D), jnp.float32),
            pltpu.SemaphoreType.DMA,
        ],
    )
    return fn(slots1, slots2, ys)


def _combine_kernel(meta_ref, z1_ref, z2_ref, out_ref):
    p1 = meta_ref[:, 0:1]
    p2 = meta_ref[:, 1:2]
    out_ref[...] = p1 * z1_ref[...] + p2 * z2_ref[...]


def _combine(meta, z1, z2):
    nb = 2
    bt = _B // nb
    return pl.pallas_call(
        _combine_kernel,
        grid=(nb,),
        in_specs=[
            pl.BlockSpec((bt, _E), lambda i: (i, 0)),
            pl.BlockSpec((bt, _D), lambda i: (i, 0)),
            pl.BlockSpec((bt, _D), lambda i: (i, 0)),
        ],
        out_specs=pl.BlockSpec((bt, _D), lambda i: (i, 0)),
        out_shape=jax.ShapeDtypeStruct((_B, _D), jnp.float32),
    )(meta, z1, z2)


def kernel(x, W_experts, b_experts, W_gate, b_gate):
    meta, be_f, cvb = _gate_route(x, W_gate, b_gate)
    slots1 = meta[:, 2].astype(jnp.int32)
    slots2 = meta[:, 3].astype(jnp.int32)
    be = be_f.reshape(_NBLK).astype(jnp.int32)
    xs = _sc_gather(slots1, slots2, x)
    ys = _group_mm(be, xs, W_experts, b_experts)
    z1, z2 = _sc_combine_gather(slots1, slots2, ys)
    out = _combine(meta, z1, z2)
    return (out, _LAMBDA * cvb[0, 0])


# speedup vs baseline: 1.4703x; 1.4703x over previous
import jax
import jax.numpy as jnp
from jax.experimental import pallas as pl
from jax.experimental.pallas import tpu as pltpu

_LAMBDA = 1.0
_NEG_INF = float("-inf")
_B = 4096
_D = 1024
_E = 8
_BT = 512
_NTB = _B // _BT
_GC = 1024


def _moe_kernel(x_ref, wg_ref, bg_ref, w_ref, be_ref, out_ref, cv_ref,
                p_scratch, xs_scratch, wbf_scratch):
    tb = pl.program_id(0)
    e = pl.program_id(1)

    @pl.when(e == 0)
    def _gate_blk():
        xv = x_ref[...]
        logits = jax.lax.dot_general(
            xv, wg_ref[...], (((1,), (1,)), ((), ())),
            preferred_element_type=jnp.float32) + bg_ref[...]
        idx8 = jax.lax.broadcasted_iota(jnp.int32, (_BT, _E), 1)
        m1 = jnp.max(logits, axis=1, keepdims=True)
        i1 = jnp.min(jnp.where(logits == m1, idx8, _E), axis=1, keepdims=True)
        sel1 = idx8 == i1
        masked = jnp.where(sel1, _NEG_INF, logits)
        m2 = jnp.max(masked, axis=1, keepdims=True)
        i2 = jnp.min(jnp.where(masked == m2, idx8, _E), axis=1, keepdims=True)
        sel2 = idx8 == i2
        e2v = jnp.exp(m2 - m1)
        z = 1.0 + e2v
        p1 = 1.0 / z
        p2 = e2v / z
        p_scratch[...] = jnp.where(sel1, p1, jnp.where(sel2, p2, 0.0))

    @pl.when(e == 0)
    def _load():
        blk_load = jnp.sum(p_scratch[...], axis=0, keepdims=True)

        @pl.when(tb == 0)
        def _():
            cv_ref[0:1, 0:8] = blk_load

        @pl.when(tb != 0)
        def _():
            cv_ref[0:1, 0:8] += blk_load

        @pl.when(tb == _NTB - 1)
        def _():
            load = cv_ref[0:1, 0:8]
            mean = jnp.sum(load) / float(_E)
            var = jnp.sum((load - mean) ** 2) / float(_E - 1)
            cv_ref[...] = jnp.full((8, 128), jnp.sqrt(var) / mean,
                                   jnp.float32)

    @pl.when(tb == 0)
    def _wconv():
        wbf_scratch[:, pl.ds(e * _D, _D)] = w_ref[0].astype(jnp.bfloat16)

    pe = jnp.sum(
        jnp.where(
            jax.lax.broadcasted_iota(jnp.int32, (_BT, _E), 1) == e,
            p_scratch[...], 0.0),
        axis=1, keepdims=True)
    xs_scratch[:, pl.ds(e * _D, _D)] = (
        x_ref[...].astype(jnp.bfloat16) * pe.astype(jnp.bfloat16))

    @pl.when(e == _E - 1)
    def _mm():
        y = jax.lax.dot_general(
            xs_scratch[...], wbf_scratch[...], (((1,), (1,)), ((), ())),
            preferred_element_type=jnp.float32)
        pb = jax.lax.dot_general(
            p_scratch[...], be_ref[...], (((1,), (0,)), ((), ())),
            preferred_element_type=jnp.float32)
        out_ref[...] = y + pb


def kernel(x, W_experts, b_experts, W_gate, b_gate):
    out, cvb = pl.pallas_call(
        _moe_kernel,
        grid=(_NTB, _E),
        in_specs=[
            pl.BlockSpec((_BT, _D), lambda tb, e: (tb, 0)),
            pl.BlockSpec((_E, _D), lambda tb, e: (0, 0)),
            pl.BlockSpec((1, _E), lambda tb, e: (0, 0)),
            pl.BlockSpec((1, _D, _D),
                         lambda tb, e: ((tb == 0).astype(jnp.int32) * e,
                                        0, 0)),
            pl.BlockSpec((_E, _D), lambda tb, e: (0, 0)),
        ],
        out_specs=[
            pl.BlockSpec((_BT, _D), lambda tb, e: (tb, 0)),
            pl.BlockSpec((8, 128), lambda tb, e: (0, 0)),
        ],
        out_shape=[
            jax.ShapeDtypeStruct((_B, _D), jnp.float32),
            jax.ShapeDtypeStruct((8, 128), jnp.float32),
        ],
        scratch_shapes=[
            pltpu.VMEM((_BT, _E), jnp.float32),
            pltpu.VMEM((_BT, _E * _D), jnp.bfloat16),
            pltpu.VMEM((_D, _E * _D), jnp.bfloat16),
        ],
    )(x, W_gate, b_gate.reshape(1, _E), W_experts, b_experts)
    return (out, _LAMBDA * cvb[0, 0])


# speedup vs baseline: 1.5376x; 1.0457x over previous
import jax
import jax.numpy as jnp
from jax.experimental import pallas as pl
from jax.experimental.pallas import tpu as pltpu

_LAMBDA = 1.0
_NEG_INF = float("-inf")
_B = 4096
_D = 1024
_E = 8
_BT = 512
_NTB = _B // _BT
_GC = 1024


def _moe_kernel(x_ref, wg_ref, bg_ref, w_ref, be_ref, out_ref, cv_ref,
                p_scratch, xs_scratch, wbf_scratch, xbf_scratch):
    tb = pl.program_id(0)
    e = pl.program_id(1)

    @pl.when(e == 0)
    def _gate_blk():
        xv = x_ref[...]
        logits = jax.lax.dot_general(
            xv, wg_ref[...], (((1,), (1,)), ((), ())),
            preferred_element_type=jnp.float32) + bg_ref[...]
        idx8 = jax.lax.broadcasted_iota(jnp.int32, (_BT, _E), 1)
        m1 = jnp.max(logits, axis=1, keepdims=True)
        i1 = jnp.min(jnp.where(logits == m1, idx8, _E), axis=1, keepdims=True)
        sel1 = idx8 == i1
        masked = jnp.where(sel1, _NEG_INF, logits)
        m2 = jnp.max(masked, axis=1, keepdims=True)
        i2 = jnp.min(jnp.where(masked == m2, idx8, _E), axis=1, keepdims=True)
        sel2 = idx8 == i2
        e2v = jnp.exp(m2 - m1)
        z = 1.0 + e2v
        p1 = 1.0 / z
        p2 = e2v / z
        p_scratch[...] = jnp.where(sel1, p1, jnp.where(sel2, p2, 0.0))
        xbf_scratch[...] = xv.astype(jnp.bfloat16)

    @pl.when(e == 0)
    def _load():
        blk_load = jnp.sum(p_scratch[...], axis=0, keepdims=True)

        @pl.when(tb == 0)
        def _():
            cv_ref[0:1, 0:8] = blk_load

        @pl.when(tb != 0)
        def _():
            cv_ref[0:1, 0:8] += blk_load

        @pl.when(tb == _NTB - 1)
        def _():
            load = cv_ref[0:1, 0:8]
            mean = jnp.sum(load) / float(_E)
            var = jnp.sum((load - mean) ** 2) / float(_E - 1)
            cv_ref[...] = jnp.full((8, 128), jnp.sqrt(var) / mean,
                                   jnp.float32)

    @pl.when(tb == 0)
    def _wconv():
        wbf_scratch[:, pl.ds(e * _D, _D)] = w_ref[0].astype(jnp.bfloat16)

    pe = jnp.sum(
        jnp.where(
            jax.lax.broadcasted_iota(jnp.int32, (_BT, _E), 1) == e,
            p_scratch[...], 0.0),
        axis=1, keepdims=True)
    xs_scratch[:, pl.ds(e * _D, _D)] = (
        xbf_scratch[...] * pe.astype(jnp.bfloat16))

    @pl.when(e == _E - 1)
    def _mm():
        y = jax.lax.dot_general(
            xs_scratch[...], wbf_scratch[...], (((1,), (1,)), ((), ())),
            preferred_element_type=jnp.float32)
        pb = jax.lax.dot_general(
            p_scratch[...], be_ref[...], (((1,), (0,)), ((), ())),
            preferred_element_type=jnp.float32)
        out_ref[...] = y + pb


def kernel(x, W_experts, b_experts, W_gate, b_gate):
    out, cvb = pl.pallas_call(
        _moe_kernel,
        grid=(_NTB, _E),
        in_specs=[
            pl.BlockSpec((_BT, _D), lambda tb, e: (tb, 0)),
            pl.BlockSpec((_E, _D), lambda tb, e: (0, 0)),
            pl.BlockSpec((1, _E), lambda tb, e: (0, 0)),
            pl.BlockSpec((1, _D, _D),
                         lambda tb, e: ((tb == 0).astype(jnp.int32) * e,
                                        0, 0)),
            pl.BlockSpec((_E, _D), lambda tb, e: (0, 0)),
        ],
        out_specs=[
            pl.BlockSpec((_BT, _D), lambda tb, e: (tb, 0)),
            pl.BlockSpec((8, 128), lambda tb, e: (0, 0)),
        ],
        out_shape=[
            jax.ShapeDtypeStruct((_B, _D), jnp.float32),
            jax.ShapeDtypeStruct((8, 128), jnp.float32),
        ],
        scratch_shapes=[
            pltpu.VMEM((_BT, _E), jnp.float32),
            pltpu.VMEM((_BT, _E * _D), jnp.bfloat16),
            pltpu.VMEM((_D, _E * _D), jnp.bfloat16),
            pltpu.VMEM((_BT, _D), jnp.bfloat16),
        ],
    )(x, W_gate, b_gate.reshape(1, _E), W_experts, b_experts)
    return (out, _LAMBDA * cvb[0, 0])


# speedup vs baseline: 1.6253x; 1.0571x over previous
import jax
import jax.numpy as jnp
from jax.experimental import pallas as pl
from jax.experimental.pallas import tpu as pltpu

_LAMBDA = 1.0
_NEG_INF = float("-inf")
_B = 4096
_D = 1024
_E = 8
_BT = 2048
_NTB = _B // _BT


def _moe_kernel(x_ref, wg_ref, bg_ref, w_ref, be_ref, out_ref, cv_ref,
                p_scratch, xbf_scratch):
    tb = pl.program_id(0)
    e = pl.program_id(1)

    @pl.when(e == 0)
    def _gate_blk():
        xv = x_ref[...]
        logits = jax.lax.dot_general(
            xv, wg_ref[...], (((1,), (1,)), ((), ())),
            preferred_element_type=jnp.float32) + bg_ref[...]
        idx8 = jax.lax.broadcasted_iota(jnp.int32, (_BT, _E), 1)
        m1 = jnp.max(logits, axis=1, keepdims=True)
        i1 = jnp.min(jnp.where(logits == m1, idx8, _E), axis=1, keepdims=True)
        sel1 = idx8 == i1
        masked = jnp.where(sel1, _NEG_INF, logits)
        m2 = jnp.max(masked, axis=1, keepdims=True)
        i2 = jnp.min(jnp.where(masked == m2, idx8, _E), axis=1, keepdims=True)
        sel2 = idx8 == i2
        e2v = jnp.exp(m2 - m1)
        z = 1.0 + e2v
        p1 = 1.0 / z
        p2 = e2v / z
        pfull = jnp.where(sel1, p1, jnp.where(sel2, p2, 0.0))
        p_scratch[...] = pfull
        xbf_scratch[...] = xv.astype(jnp.bfloat16)

        blk_load = jnp.sum(pfull, axis=0, keepdims=True)

        @pl.when(tb == 0)
        def _():
            cv_ref[0:1, 0:8] = blk_load

        @pl.when(tb != 0)
        def _():
            cv_ref[0:1, 0:8] += blk_load

        @pl.when(tb == _NTB - 1)
        def _():
            load = cv_ref[0:1, 0:8]
            mean = jnp.sum(load) / float(_E)
            var = jnp.sum((load - mean) ** 2) / float(_E - 1)
            cv_ref[...] = jnp.full((8, 128), jnp.sqrt(var) / mean,
                                   jnp.float32)

    pe = jnp.sum(
        jnp.where(
            jax.lax.broadcasted_iota(jnp.int32, (_BT, _E), 1) == e,
            p_scratch[...], 0.0),
        axis=1, keepdims=True)
    xs = xbf_scratch[...] * pe.astype(jnp.bfloat16)
    wb = w_ref[0].astype(jnp.bfloat16)
    y = jax.lax.dot_general(
        xs, wb, (((1,), (1,)), ((), ())),
        preferred_element_type=jnp.float32)

    @pl.when(e == 0)
    def _init():
        out_ref[...] = y

    @pl.when((e != 0) & (e != _E - 1))
    def _acc():
        out_ref[...] += y

    @pl.when(e == _E - 1)
    def _last():
        pb = jax.lax.dot_general(
            p_scratch[...], be_ref[...], (((1,), (0,)), ((), ())),
            preferred_element_type=jnp.float32)
        out_ref[...] += y + pb


def kernel(x, W_experts, b_experts, W_gate, b_gate):
    out, cvb = pl.pallas_call(
        _moe_kernel,
        grid=(_NTB, _E),
        in_specs=[
            pl.BlockSpec((_BT, _D), lambda tb, e: (tb, 0)),
            pl.BlockSpec((_E, _D), lambda tb, e: (0, 0)),
            pl.BlockSpec((1, _E), lambda tb, e: (0, 0)),
            pl.BlockSpec((1, _D, _D), lambda tb, e: (e, 0, 0)),
            pl.BlockSpec((_E, _D), lambda tb, e: (0, 0)),
        ],
        out_specs=[
            pl.BlockSpec((_BT, _D), lambda tb, e: (tb, 0)),
            pl.BlockSpec((8, 128), lambda tb, e: (0, 0)),
        ],
        out_shape=[
            jax.ShapeDtypeStruct((_B, _D), jnp.float32),
            jax.ShapeDtypeStruct((8, 128), jnp.float32),
        ],
        scratch_shapes=[
            pltpu.VMEM((_BT, _E), jnp.float32),
            pltpu.VMEM((_BT, _D), jnp.bfloat16),
        ],
    )(x, W_gate, b_gate.reshape(1, _E), W_experts, b_experts)
    return (out, _LAMBDA * cvb[0, 0])
